# baseline (speedup 1.0000x reference)
import jax
import jax.numpy as jnp
from jax import lax
from jax.experimental import pallas as pl
from jax.experimental.pallas import tpu as pltpu
from jax.experimental.pallas import tpu_sc as plsc

_VOCAB = 500
_EMBED = 10
_H1 = 16
_H2 = 32
_ACT = 6
_NEG = -1e30


def _policy_body(x_hbm, tab_hbm, w1_hbm, b1_hbm, w2_hbm, b2_hbm, w3_hbm,
                 b3_hbm, out_hbm, xs, tab_v, w1_v, b1_v, w2_v, b2_v, w3_v,
                 b3_v, out6, sem):
    is_worker = (lax.axis_index("c") == 0) & (lax.axis_index("s") == 0)

    @pl.when(is_worker)
    def _():
        copies = [
            pltpu.async_copy(x_hbm, xs.at[pl.ds(0, 1)], sem),
            pltpu.async_copy(tab_hbm, tab_v.at[pl.ds(0, _VOCAB * _EMBED)],
                             sem),
            pltpu.async_copy(w1_hbm, w1_v, sem),
            pltpu.async_copy(b1_hbm, b1_v, sem),
            pltpu.async_copy(w2_hbm, w2_v, sem),
            pltpu.async_copy(b2_hbm, b2_v, sem),
            pltpu.async_copy(w3_hbm, w3_v.at[pl.ds(0, _H2 * _ACT)], sem),
            pltpu.async_copy(b3_hbm, b3_v.at[pl.ds(0, _ACT)], sem),
        ]
        for c in copies:
            c.wait()

        iota = jax.lax.iota(jnp.int32, 16)
        idx = xs[...][0]
        ev = tab_v[pl.ds(idx * _EMBED, 16)]

        h1 = b1_v[...]
        for i in range(_EMBED):
            h1 = h1 + ev[i] * w1_v[i, :]
        h1 = jnp.maximum(h1, 0.0)

        h2a = b2_v[pl.ds(0, 16)]
        h2b = b2_v[pl.ds(16, 16)]
        for i in range(_H1):
            s = h1[i]
            h2a = h2a + s * w2_v[i, pl.ds(0, 16)]
            h2b = h2b + s * w2_v[i, pl.ds(16, 16)]
        h2a = jnp.maximum(h2a, 0.0)
        h2b = jnp.maximum(h2b, 0.0)

        acc = b3_v[...]
        for i in range(_H1):
            acc = acc + h2a[i] * w3_v[pl.ds(i * _ACT, 16)]
        for i in range(_H1, _H2):
            acc = acc + h2b[i - _H1] * w3_v[pl.ds(i * _ACT, 16)]
        logits = jnp.where(iota < _ACT, acc, _NEG)

        m = logits[0]
        for j in range(1, _ACT):
            m = jnp.maximum(m, logits[j])
        e = jnp.exp(logits - m)
        s = e[0]
        for j in range(1, _ACT):
            s = s + e[j]
        p = e / s
        out6[...] = p
        pltpu.async_copy(out6.at[pl.ds(0, _ACT)], out_hbm, sem).wait()


@jax.jit
def _policy_sc(x, table, W1, b1, W2, b2, W3, b3):
    mesh = plsc.VectorSubcoreMesh(core_axis_name="c", subcore_axis_name="s")
    f = pl.kernel(
        _policy_body,
        out_type=jax.ShapeDtypeStruct((_ACT,), jnp.float32),
        mesh=mesh,
        scratch_types=[
            pltpu.VMEM((16,), jnp.int32),
            pltpu.VMEM((_VOCAB * _EMBED + 16,), jnp.float32),
            pltpu.VMEM((_EMBED, _H1), jnp.float32),
            pltpu.VMEM((_H1,), jnp.float32),
            pltpu.VMEM((_H1, _H2), jnp.float32),
            pltpu.VMEM((_H2,), jnp.float32),
            pltpu.VMEM((_H2 * _ACT + 16,), jnp.float32),
            pltpu.VMEM((16,), jnp.float32),
            pltpu.VMEM((16,), jnp.float32),
            pltpu.SemaphoreType.DMA,
        ],
    )
    out = f(x.astype(jnp.int32), table.reshape(-1), W1, b1, W2, b2,
            W3.reshape(-1), b3)
    return out.reshape(1, _ACT)


def kernel(x, table, W1, b1, W2, b2, W3, b3):
    return _policy_sc(x, table, W1, b1, W2, b2, W3, b3)


# speedup vs baseline: 1.1079x; 1.1079x over previous
import jax
import jax.numpy as jnp
from jax import lax
from jax.experimental import pallas as pl
from jax.experimental.pallas import tpu as pltpu
from jax.experimental.pallas import tpu_sc as plsc

_VOCAB = 500
_EMBED = 10
_H1 = 16
_H2 = 32
_ACT = 6
_NEG = -1e30


def _policy_body(x_hbm, tab_hbm, w1_hbm, b1_hbm, w2_hbm, b2_hbm, w3_hbm,
                 b3_hbm, out_hbm, xs, tab_v, w1_v, b1_v, w2_v, b2_v, w3_v,
                 b3_v, out6, sem):
    is_worker = (lax.axis_index("c") == 0) & (lax.axis_index("s") == 0)

    @pl.when(is_worker)
    def _():
        copies = [
            pltpu.async_copy(x_hbm, xs.at[pl.ds(0, 1)], sem),
            pltpu.async_copy(tab_hbm, tab_v.at[pl.ds(0, _VOCAB * _EMBED)],
                             sem),
            pltpu.async_copy(w1_hbm, w1_v, sem),
            pltpu.async_copy(b1_hbm, b1_v, sem),
            pltpu.async_copy(w2_hbm, w2_v, sem),
            pltpu.async_copy(b2_hbm, b2_v, sem),
            pltpu.async_copy(w3_hbm, w3_v.at[pl.ds(0, _H2 * _ACT)], sem),
            pltpu.async_copy(b3_hbm, b3_v.at[pl.ds(0, _ACT)], sem),
        ]
        for c in copies:
            c.wait()

        iota = jax.lax.iota(jnp.int32, 16)
        idx = xs[...][0]
        ev = tab_v[pl.ds(idx * _EMBED, 16)]

        h1 = b1_v[...]
        for i in range(_EMBED):
            h1 = h1 + ev[i] * w1_v[i, :]
        h1 = jnp.maximum(h1, 0.0)

        h2a = b2_v[pl.ds(0, 16)]
        h2b = b2_v[pl.ds(16, 16)]
        for i in range(_H1):
            s = h1[i]
            h2a = h2a + s * w2_v[i, pl.ds(0, 16)]
            h2b = h2b + s * w2_v[i, pl.ds(16, 16)]
        h2a = jnp.maximum(h2a, 0.0)
        h2b = jnp.maximum(h2b, 0.0)

        acc = b3_v[...]
        for i in range(_H1):
            acc = acc + h2a[i] * w3_v[pl.ds(i * _ACT, 16)]
        for i in range(_H1, _H2):
            acc = acc + h2b[i - _H1] * w3_v[pl.ds(i * _ACT, 16)]
        logits = jnp.where(iota < _ACT, acc, _NEG)

        m = logits[0]
        for j in range(1, _ACT):
            m = jnp.maximum(m, logits[j])
        e = jnp.exp(logits - m)
        s = e[0]
        for j in range(1, _ACT):
            s = s + e[j]
        p = e / s
        out6[...] = p
        pltpu.async_copy(out6.at[pl.ds(0, _ACT)], out_hbm, sem).wait()


@jax.jit
def _policy_sc(x, table, W1, b1, W2, b2, W3, b3):
    mesh = plsc.VectorSubcoreMesh(core_axis_name="c", subcore_axis_name="s",
                                  num_cores=1)
    f = pl.kernel(
        _policy_body,
        out_type=jax.ShapeDtypeStruct((_ACT,), jnp.float32),
        mesh=mesh,
        scratch_types=[
            pltpu.VMEM((16,), jnp.int32),
            pltpu.VMEM((_VOCAB * _EMBED + 16,), jnp.float32),
            pltpu.VMEM((_EMBED, _H1), jnp.float32),
            pltpu.VMEM((_H1,), jnp.float32),
            pltpu.VMEM((_H1, _H2), jnp.float32),
            pltpu.VMEM((_H2,), jnp.float32),
            pltpu.VMEM((_H2 * _ACT + 16,), jnp.float32),
            pltpu.VMEM((16,), jnp.float32),
            pltpu.VMEM((16,), jnp.float32),
            pltpu.SemaphoreType.DMA,
        ],
    )
    out = f(x.astype(jnp.int32), table.reshape(-1), W1, b1, W2, b2,
            W3.reshape(-1), b3)
    return out.reshape(1, _ACT)


def kernel(x, table, W1, b1, W2, b2, W3, b3):
    return _policy_sc(x, table, W1, b1, W2, b2, W3, b3)


# speedup vs baseline: 1.1136x; 1.0051x over previous
import jax
import jax.numpy as jnp
from jax import lax
from jax.experimental import pallas as pl
from jax.experimental.pallas import tpu as pltpu
from jax.experimental.pallas import tpu_sc as plsc

_VOCAB = 500
_EMBED = 10
_H1 = 16
_H2 = 32
_ACT = 6
_NEG = -1e30


def _policy_body(x_hbm, tab_hbm, w1_hbm, b1_hbm, w2_hbm, b2_hbm, w3_hbm,
                 b3_hbm, out_hbm, xs, tab_v, w1_v, b1_v, w2_v, b2_v, w3_v,
                 b3_v, out6, sem):
    is_worker = (lax.axis_index("c") == 0) & (lax.axis_index("s") == 0)

    @pl.when(is_worker)
    def _():
        copies = [
            pltpu.async_copy(x_hbm, xs.at[pl.ds(0, 1)], sem),
            pltpu.async_copy(tab_hbm, tab_v.at[pl.ds(0, _VOCAB * _EMBED)],
                             sem),
            pltpu.async_copy(w1_hbm, w1_v, sem),
            pltpu.async_copy(b1_hbm, b1_v, sem),
            pltpu.async_copy(w2_hbm, w2_v, sem),
            pltpu.async_copy(b2_hbm, b2_v, sem),
            pltpu.async_copy(w3_hbm, w3_v.at[pl.ds(0, _H2 * _ACT)], sem),
            pltpu.async_copy(b3_hbm, b3_v.at[pl.ds(0, _ACT)], sem),
        ]
        for c in copies:
            c.wait()

        iota = jax.lax.iota(jnp.int32, 16)
        idx = xs[...][0]
        ev = tab_v[pl.ds(idx * _EMBED, 16)]

        h1 = b1_v[...]
        for i in range(_EMBED):
            h1 = h1 + ev[i] * w1_v[i, :]
        h1 = jnp.maximum(h1, 0.0)

        h2a = b2_v[pl.ds(0, 16)]
        h2b = b2_v[pl.ds(16, 16)]
        for i in range(_H1):
            s = h1[i]
            h2a = h2a + s * w2_v[i, pl.ds(0, 16)]
            h2b = h2b + s * w2_v[i, pl.ds(16, 16)]
        h2a = jnp.maximum(h2a, 0.0)
        h2b = jnp.maximum(h2b, 0.0)

        acc = b3_v[...]
        for i in range(_H1):
            acc = acc + h2a[i] * w3_v[pl.ds(i * _ACT, 16)]
        for i in range(_H1, _H2):
            acc = acc + h2b[i - _H1] * w3_v[pl.ds(i * _ACT, 16)]
        logits = jnp.where(iota < _ACT, acc, _NEG)

        m = logits[0]
        for j in range(1, _ACT):
            m = jnp.maximum(m, logits[j])
        e = jnp.exp(logits - m)
        s = e[0]
        for j in range(1, _ACT):
            s = s + e[j]
        p = e / s
        out6[...] = p
        pltpu.async_copy(out6.at[pl.ds(0, _ACT)], out_hbm, sem).wait()


@jax.jit
def _policy_sc(x, table, W1, b1, W2, b2, W3, b3):
    mesh = plsc.VectorSubcoreMesh(core_axis_name="c", subcore_axis_name="s",
                                  num_cores=1, num_subcores=1)
    f = pl.kernel(
        _policy_body,
        out_type=jax.ShapeDtypeStruct((_ACT,), jnp.float32),
        mesh=mesh,
        scratch_types=[
            pltpu.VMEM((16,), jnp.int32),
            pltpu.VMEM((_VOCAB * _EMBED + 16,), jnp.float32),
            pltpu.VMEM((_EMBED, _H1), jnp.float32),
            pltpu.VMEM((_H1,), jnp.float32),
            pltpu.VMEM((_H1, _H2), jnp.float32),
            pltpu.VMEM((_H2,), jnp.float32),
            pltpu.VMEM((_H2 * _ACT + 16,), jnp.float32),
            pltpu.VMEM((16,), jnp.float32),
            pltpu.VMEM((16,), jnp.float32),
            pltpu.SemaphoreType.DMA,
        ],
    )
    out = f(x.astype(jnp.int32), table.reshape(-1), W1, b1, W2, b2,
            W3.reshape(-1), b3)
    return out.reshape(1, _ACT)


def kernel(x, table, W1, b1, W2, b2, W3, b3):
    return _policy_sc(x, table, W1, b1, W2, b2, W3, b3)
